# baseline (speedup 1.0000x reference)
import functools
import math

import jax
import jax.numpy as jnp
from jax.experimental import pallas as pl
from jax.experimental.pallas import tpu as pltpu

_RATIO = 0.6


def _mm_kern(x_ref, w_ref, b_ref, o_ref, acc_ref, *, relu):
    @pl.when(pl.program_id(2) == 0)
    def _():
        acc_ref[...] = jnp.zeros_like(acc_ref)

    acc_ref[...] += jnp.dot(x_ref[...], w_ref[...],
                            preferred_element_type=jnp.float32)

    @pl.when(pl.program_id(2) == pl.num_programs(2) - 1)
    def _():
        r = acc_ref[...] + b_ref[...]
        if relu:
            r = jnp.maximum(r, 0.0)
        o_ref[...] = r


def _matmul_bias(x, W, b, relu=False, bm=1000, bn=512, bk=512):
    M, K = x.shape
    _, N = W.shape
    assert M % bm == 0 and N % bn == 0 and K % bk == 0, (M, N, K)
    return pl.pallas_call(
        functools.partial(_mm_kern, relu=relu),
        grid=(M // bm, N // bn, K // bk),
        in_specs=[
            pl.BlockSpec((bm, bk), lambda i, j, k: (i, k)),
            pl.BlockSpec((bk, bn), lambda i, j, k: (k, j)),
            pl.BlockSpec((1, bn), lambda i, j, k: (0, j)),
        ],
        out_specs=pl.BlockSpec((bm, bn), lambda i, j, k: (i, j)),
        out_shape=jax.ShapeDtypeStruct((M, N), jnp.float32),
        scratch_shapes=[pltpu.VMEM((bm, bn), jnp.float32)],
        compiler_params=pltpu.CompilerParams(
            dimension_semantics=("parallel", "parallel", "arbitrary")),
    )(x, W, b.reshape(1, N))


def _mlp_kern(p_ref, w1_ref, b1_ref, w2_ref, b2_ref, w3_ref, b3_ref, o_ref):
    h = jnp.dot(p_ref[...], w1_ref[...], preferred_element_type=jnp.float32)
    h = jnp.maximum(h + b1_ref[...], 0.0)
    h = jnp.dot(h, w2_ref[...], preferred_element_type=jnp.float32)
    h = jnp.maximum(h + b2_ref[...], 0.0)
    h = jnp.dot(h, w3_ref[...], preferred_element_type=jnp.float32)
    o_ref[...] = jax.nn.sigmoid(h + b3_ref[...])


def _mlp_head(pooled, w1, b1, w2, b2, w3, b3):
    d = pooled.shape[0]
    h1 = w1.shape[1]
    c = w3.shape[1]
    cp = ((c + 127) // 128) * 128
    w3p = jnp.pad(w3, ((0, 0), (0, cp - c)))
    b3p = jnp.pad(b3, (0, cp - c))
    pin = jnp.zeros((8, d), jnp.float32).at[0].set(pooled)
    out = pl.pallas_call(
        _mlp_kern,
        out_shape=jax.ShapeDtypeStruct((8, cp), jnp.float32),
    )(pin, w1, b1.reshape(1, h1), w2, b2.reshape(1, h1),
      w3p, b3p.reshape(1, cp))
    return out[0, :c]


def kernel(x, edge_index, W1, b1, W2, b2, W3, b3, p1, p2, p3,
           lin1_W, lin1_b, lin2_W, lin2_b, lin3_W, lin3_b):
    n0 = x.shape[0]
    src = edge_index[0]
    dst = edge_index[1]
    mask = jnp.ones(src.shape, jnp.float32)
    alive = jnp.ones((n0,), jnp.bool_)
    zero_b = jnp.zeros((x.shape[1],), jnp.float32)

    n_cur = n0
    for (W, b, p) in ((W1, b1, p1), (W2, b2, p2), (W3, b3, p3)):
        xw = _matmul_bias(x, W, zero_b)
        deg = jnp.zeros((n0,), jnp.float32).at[dst].add(mask) + 1.0
        dinv = jax.lax.rsqrt(deg)
        norm = dinv[src] * dinv[dst] * mask
        agg = jnp.zeros_like(xw).at[dst].add(norm[:, None] * xw[src])
        h = agg + (dinv * dinv)[:, None] * xw + b
        h = jnp.maximum(h, 0.0)
        score = (h @ p) / jnp.linalg.norm(p)
        score = jnp.where(alive, score, -jnp.inf)
        k = int(math.ceil(_RATIO * n_cur))
        _, perm = jax.lax.top_k(score, k)
        alive = jnp.zeros((n0,), jnp.bool_).at[perm].set(True)
        x = h * jnp.where(alive, jnp.tanh(score), 0.0)[:, None]
        mask = mask * alive[src].astype(mask.dtype) * alive[dst].astype(mask.dtype)
        n_cur = k

    pooled = jnp.sum(x, axis=0) / jnp.float32(n_cur)
    return _mlp_head(pooled, lin1_W, lin1_b, lin2_W, lin2_b, lin3_W, lin3_b)


# speedup vs baseline: 1.0316x; 1.0316x over previous
import functools
import math

import jax
import jax.numpy as jnp
from jax.experimental import pallas as pl
from jax.experimental.pallas import tpu as pltpu

_RATIO = 0.6
_INT_MIN = -2147483648


def _mm_kern(x_ref, w_ref, b_ref, o_ref, acc_ref, *, relu):
    @pl.when(pl.program_id(2) == 0)
    def _():
        acc_ref[...] = jnp.zeros_like(acc_ref)

    acc_ref[...] += jnp.dot(x_ref[...], w_ref[...],
                            preferred_element_type=jnp.float32)

    @pl.when(pl.program_id(2) == pl.num_programs(2) - 1)
    def _():
        r = acc_ref[...] + b_ref[...]
        if relu:
            r = jnp.maximum(r, 0.0)
        o_ref[...] = r


def _pick_bm(M):
    for bm in (1000, 800, 600, 400, 200, 80, 8):
        if M % bm == 0:
            return bm
    raise ValueError(M)


def _matmul_bias(x, W, b, relu=False, bn=512, bk=512):
    M, K = x.shape
    _, N = W.shape
    bm = _pick_bm(M)
    assert N % bn == 0 and K % bk == 0, (M, N, K)
    return pl.pallas_call(
        functools.partial(_mm_kern, relu=relu),
        grid=(M // bm, N // bn, K // bk),
        in_specs=[
            pl.BlockSpec((bm, bk), lambda i, j, k: (i, k)),
            pl.BlockSpec((bk, bn), lambda i, j, k: (k, j)),
            pl.BlockSpec((1, bn), lambda i, j, k: (0, j)),
        ],
        out_specs=pl.BlockSpec((bm, bn), lambda i, j, k: (i, j)),
        out_shape=jax.ShapeDtypeStruct((M, N), jnp.float32),
        scratch_shapes=[pltpu.VMEM((bm, bn), jnp.float32)],
        compiler_params=pltpu.CompilerParams(
            dimension_semantics=("parallel", "parallel", "arbitrary")),
    )(x, W, b.reshape(1, N))


def _kth_kern(s_ref, t_ref, *, k):
    def body(i, pu):
        cu = pu | (jnp.int32(1) << (jnp.int32(31) - i))
        cand = cu ^ jnp.int32(_INT_MIN)
        cnt = jnp.sum((s_ref[...] >= cand).astype(jnp.int32))
        return jax.lax.select(cnt >= k, cu, pu)

    pu = jax.lax.fori_loop(0, 32, body, jnp.int32(0))
    t_ref[...] = (pu ^ jnp.int32(_INT_MIN)).reshape(1, 1)


def _kth_largest_key(skey2d, k):
    t = pl.pallas_call(
        functools.partial(_kth_kern, k=k),
        out_shape=jax.ShapeDtypeStruct((1, 1), jnp.int32),
    )(skey2d)
    return t[0, 0]


def _float_key(score):
    i = jax.lax.bitcast_convert_type(score, jnp.int32)
    return jnp.where(i >= 0, i, ~(i & 0x7FFFFFFF))


def _mlp_kern(p_ref, w1_ref, b1_ref, w2_ref, b2_ref, w3_ref, b3_ref, o_ref):
    h = jnp.dot(p_ref[...], w1_ref[...], preferred_element_type=jnp.float32)
    h = jnp.maximum(h + b1_ref[...], 0.0)
    h = jnp.dot(h, w2_ref[...], preferred_element_type=jnp.float32)
    h = jnp.maximum(h + b2_ref[...], 0.0)
    h = jnp.dot(h, w3_ref[...], preferred_element_type=jnp.float32)
    o_ref[...] = jax.nn.sigmoid(h + b3_ref[...])


def _mlp_head(pooled, w1, b1, w2, b2, w3, b3):
    d = pooled.shape[0]
    h1 = w1.shape[1]
    c = w3.shape[1]
    cp = ((c + 127) // 128) * 128
    w3p = jnp.pad(w3, ((0, 0), (0, cp - c)))
    b3p = jnp.pad(b3, (0, cp - c))
    pin = jnp.zeros((8, d), jnp.float32).at[0].set(pooled)
    out = pl.pallas_call(
        _mlp_kern,
        out_shape=jax.ShapeDtypeStruct((8, cp), jnp.float32),
    )(pin, w1, b1.reshape(1, h1), w2, b2.reshape(1, h1),
      w3p, b3p.reshape(1, cp))
    return out[0, :c]


def kernel(x, edge_index, W1, b1, W2, b2, W3, b3, p1, p2, p3,
           lin1_W, lin1_b, lin2_W, lin2_b, lin3_W, lin3_b):
    src = edge_index[0]
    dst = edge_index[1]
    mask = jnp.ones(src.shape, jnp.float32)
    n_cur = x.shape[0]

    for (W, b, p) in ((W1, b1, p1), (W2, b2, p2), (W3, b3, p3)):
        xw = _matmul_bias(x, W, jnp.zeros((x.shape[1],), jnp.float32))
        deg = jnp.zeros((n_cur,), jnp.float32).at[dst].add(mask) + 1.0
        dinv = jax.lax.rsqrt(deg)
        norm = dinv[src] * dinv[dst] * mask
        h = jnp.zeros_like(xw).at[dst].add(norm[:, None] * xw[src])
        h = h + (dinv * dinv)[:, None] * xw + b
        h = jnp.maximum(h, 0.0)
        score = (h @ p) / jnp.linalg.norm(p)

        k = int(math.ceil(_RATIO * n_cur))
        skey = _float_key(score)
        m = ((n_cur + 127) // 128) * 128
        skey_pad = jnp.full((m,), _INT_MIN, dtype=jnp.int32).at[:n_cur].set(skey)
        t = _kth_largest_key(skey_pad.reshape(m // 128, 128), k)
        n_gt = jnp.sum((skey > t).astype(jnp.int32))
        eq = skey == t
        sel = (skey > t) | (eq & (jnp.cumsum(eq.astype(jnp.int32)) <= k - n_gt))
        newid = jnp.cumsum(sel.astype(jnp.int32)) - 1
        keep = (jnp.zeros((k,), jnp.int32)
                .at[jnp.where(sel, newid, k)]
                .set(jnp.arange(n_cur, dtype=jnp.int32), mode="drop"))
        gate = jnp.where(sel, jnp.tanh(score), 0.0)
        x = h[keep] * gate[keep][:, None]
        mask = mask * sel[src].astype(mask.dtype) * sel[dst].astype(mask.dtype)
        src = jnp.clip(newid[src], 0, k - 1).astype(jnp.int32)
        dst = jnp.clip(newid[dst], 0, k - 1).astype(jnp.int32)
        n_cur = k

    pooled = jnp.sum(x, axis=0) / jnp.float32(n_cur)
    return _mlp_head(pooled, lin1_W, lin1_b, lin2_W, lin2_b, lin3_W, lin3_b)


# speedup vs baseline: 1.0903x; 1.0569x over previous
import functools
import math

import jax
import jax.numpy as jnp
from jax.experimental import pallas as pl
from jax.experimental.pallas import tpu as pltpu

_RATIO = 0.6
_INT_MIN = -2147483648


def _mm_kern(x_ref, g_ref, w_ref, o_ref, acc_ref):
    @pl.when(pl.program_id(2) == 0)
    def _():
        acc_ref[...] = jnp.zeros_like(acc_ref)

    acc_ref[...] += jnp.dot(x_ref[...] * g_ref[...], w_ref[...],
                            preferred_element_type=jnp.float32)

    @pl.when(pl.program_id(2) == pl.num_programs(2) - 1)
    def _():
        o_ref[...] = acc_ref[...]


def _pick_bm(M):
    for bm in (1000, 800, 600, 400, 200, 80, 8):
        if M % bm == 0:
            return bm
    raise ValueError(M)


def _matmul_gated(x, gate2d, W, bn=512, bk=512):
    M, K = x.shape
    _, N = W.shape
    bm = _pick_bm(M)
    assert N % bn == 0 and K % bk == 0, (M, N, K)
    return pl.pallas_call(
        _mm_kern,
        grid=(M // bm, N // bn, K // bk),
        in_specs=[
            pl.BlockSpec((bm, bk), lambda i, j, k: (i, k)),
            pl.BlockSpec((bm, 1), lambda i, j, k: (i, 0)),
            pl.BlockSpec((bk, bn), lambda i, j, k: (k, j)),
        ],
        out_specs=pl.BlockSpec((bm, bn), lambda i, j, k: (i, j)),
        out_shape=jax.ShapeDtypeStruct((M, N), jnp.float32),
        scratch_shapes=[pltpu.VMEM((bm, bn), jnp.float32)],
        compiler_params=pltpu.CompilerParams(
            dimension_semantics=("parallel", "parallel", "arbitrary")),
    )(x, gate2d, W)


def _epi_kern(agg_ref, xw_ref, dinv_ref, b_ref, p_ref, h_ref, sp_ref):
    d2 = dinv_ref[...] * dinv_ref[...]
    h = agg_ref[...] + d2 * xw_ref[...] + b_ref[...]
    h = jnp.maximum(h, 0.0)
    h_ref[...] = h
    sp_ref[...] = jnp.sum(h * p_ref[...], axis=1).reshape(1, 1, -1)


def _conv_epilogue(agg, xw, dinv2d, b, p, bm=400):
    M, N = agg.shape
    assert M % bm == 0
    h, sp = pl.pallas_call(
        _epi_kern,
        grid=(M // bm,),
        in_specs=[
            pl.BlockSpec((bm, N), lambda i: (i, 0)),
            pl.BlockSpec((bm, N), lambda i: (i, 0)),
            pl.BlockSpec((bm, 1), lambda i: (i, 0)),
            pl.BlockSpec((1, N), lambda i: (0, 0)),
            pl.BlockSpec((1, N), lambda i: (0, 0)),
        ],
        out_specs=[
            pl.BlockSpec((bm, N), lambda i: (i, 0)),
            pl.BlockSpec((1, 1, bm), lambda i: (i, 0, 0)),
        ],
        out_shape=[
            jax.ShapeDtypeStruct((M, N), jnp.float32),
            jax.ShapeDtypeStruct((M // bm, 1, bm), jnp.float32),
        ],
    )(agg, xw, dinv2d, b.reshape(1, N), p.reshape(1, N))
    return h, sp.reshape(M)


def _sel_kern(sp_ref, alive_ref, p_ref, gate_ref, alivenew_ref, *, k, rows):
    pnorm = jnp.sqrt(jnp.sum(p_ref[...] * p_ref[...]))
    score = sp_ref[...] / pnorm
    alive = alive_ref[...] > 0.0
    i = jax.lax.bitcast_convert_type(score, jnp.int32)
    skey = jnp.where(i >= 0, i, ~(i & 0x7FFFFFFF))
    skey = jnp.where(alive, skey, jnp.int32(_INT_MIN))

    def body(it, pu):
        cu = pu | (jnp.int32(1) << (jnp.int32(31) - it))
        cand = cu ^ jnp.int32(_INT_MIN)
        cnt = jnp.sum((skey >= cand).astype(jnp.int32))
        return jax.lax.select(cnt >= k, cu, pu)

    t = jax.lax.fori_loop(0, 32, body, jnp.int32(0)) ^ jnp.int32(_INT_MIN)

    gt = skey > t
    eq = skey == t
    n_gt = jnp.sum(gt.astype(jnp.int32))
    eqf = eq.astype(jnp.float32)
    lane_i = jax.lax.broadcasted_iota(jnp.int32, (128, 128), 0)
    lane_j = jax.lax.broadcasted_iota(jnp.int32, (128, 128), 1)
    tri_incl = (lane_i <= lane_j).astype(jnp.float32)
    rowcum = jnp.dot(eqf, tri_incl, preferred_element_type=jnp.float32)
    row_i = jax.lax.broadcasted_iota(jnp.int32, (rows, rows), 0)
    row_j = jax.lax.broadcasted_iota(jnp.int32, (rows, rows), 1)
    tri_strict = (row_j < row_i).astype(jnp.float32)
    carry = jnp.dot(tri_strict, rowcum[:, 127:128],
                    preferred_element_type=jnp.float32)
    cum = (rowcum + carry).astype(jnp.int32)
    sel = gt | (eq & (cum <= (k - n_gt)))
    gate_ref[...] = jnp.where(sel, jnp.tanh(score), 0.0)
    alivenew_ref[...] = sel.astype(jnp.float32)


def _select_topk(sp_pad, alive_pad, p, k):
    rows = sp_pad.shape[0]
    return pl.pallas_call(
        functools.partial(_sel_kern, k=k, rows=rows),
        out_shape=[jax.ShapeDtypeStruct((rows, 128), jnp.float32),
                   jax.ShapeDtypeStruct((rows, 128), jnp.float32)],
    )(sp_pad, alive_pad, p.reshape(16, 128))


def _mlp_kern(p_ref, w1_ref, b1_ref, w2_ref, b2_ref, w3_ref, b3_ref, o_ref):
    h = jnp.dot(p_ref[...], w1_ref[...], preferred_element_type=jnp.float32)
    h = jnp.maximum(h + b1_ref[...], 0.0)
    h = jnp.dot(h, w2_ref[...], preferred_element_type=jnp.float32)
    h = jnp.maximum(h + b2_ref[...], 0.0)
    h = jnp.dot(h, w3_ref[...], preferred_element_type=jnp.float32)
    o_ref[...] = jax.nn.sigmoid(h + b3_ref[...])


def _mlp_head(pooled1, w1, b1, w2, b2, w3, b3):
    h1 = w1.shape[1]
    c = w3.shape[1]
    cp = ((c + 127) // 128) * 128
    w3p = jnp.pad(w3, ((0, 0), (0, cp - c)))
    b3p = jnp.pad(b3, (0, cp - c))
    out = pl.pallas_call(
        _mlp_kern,
        out_shape=jax.ShapeDtypeStruct((1, cp), jnp.float32),
    )(pooled1, w1, b1.reshape(1, h1), w2, b2.reshape(1, h1),
      w3p, b3p.reshape(1, cp))
    return out[0, :c]


def _pool_kern(g_ref, h_ref, o_ref, acc_ref, *, inv_k):
    @pl.when(pl.program_id(0) == 0)
    def _():
        acc_ref[...] = jnp.zeros_like(acc_ref)

    acc_ref[...] += jnp.sum(h_ref[...] * g_ref[...], axis=0, keepdims=True)

    @pl.when(pl.program_id(0) == pl.num_programs(0) - 1)
    def _():
        o_ref[...] = acc_ref[...] * inv_k


def _gated_pool(gate2d, h, k, bm=400):
    M, N = h.shape
    assert M % bm == 0
    return pl.pallas_call(
        functools.partial(_pool_kern, inv_k=1.0 / k),
        grid=(M // bm,),
        in_specs=[
            pl.BlockSpec((bm, 1), lambda i: (i, 0)),
            pl.BlockSpec((bm, N), lambda i: (i, 0)),
        ],
        out_specs=pl.BlockSpec((1, N), lambda i: (0, 0)),
        out_shape=jax.ShapeDtypeStruct((1, N), jnp.float32),
        scratch_shapes=[pltpu.VMEM((1, N), jnp.float32)],
        compiler_params=pltpu.CompilerParams(
            dimension_semantics=("arbitrary",)),
    )(gate2d, h)


def kernel(x, edge_index, W1, b1, W2, b2, W3, b3, p1, p2, p3,
           lin1_W, lin1_b, lin2_W, lin2_b, lin3_W, lin3_b):
    n0, d = x.shape
    src = edge_index[0]
    dst = edge_index[1]
    mask = jnp.ones(src.shape, jnp.float32)
    rows = (n0 + 127) // 128

    h = x
    gate2d = jnp.ones((n0, 1), jnp.float32)
    alive_pad = jnp.zeros((rows * 128,), jnp.float32).at[:n0].set(1.0)
    n_cur = n0
    for (W, b, p) in ((W1, b1, p1), (W2, b2, p2), (W3, b3, p3)):
        xw = _matmul_gated(h, gate2d, W)
        deg = jnp.zeros((n0,), jnp.float32).at[dst].add(mask) + 1.0
        dinv = jax.lax.rsqrt(deg)
        norm = dinv[src] * dinv[dst] * mask
        agg = jnp.zeros_like(xw).at[dst].add(norm[:, None] * xw[src])
        h, sp = _conv_epilogue(agg, xw, dinv[:, None], b, p)

        k = int(math.ceil(_RATIO * n_cur))
        sp_pad = jnp.zeros((rows * 128,), jnp.float32).at[:n0].set(sp)
        gate_pad, alive_pad2d = _select_topk(
            sp_pad.reshape(rows, 128), alive_pad.reshape(rows, 128), p, k)
        gate = gate_pad.reshape(rows * 128)[:n0]
        alive_pad = alive_pad2d.reshape(rows * 128)
        sel = alive_pad[:n0]
        mask = mask * sel[src] * sel[dst]
        gate2d = gate[:, None]
        n_cur = k

    pooled1 = _gated_pool(gate2d, h, n_cur)
    return _mlp_head(pooled1, lin1_W, lin1_b, lin2_W, lin2_b, lin3_W, lin3_b)
